# baseline (speedup 1.0000x reference)
import functools

import jax
import jax.numpy as jnp
from jax import lax
from jax.experimental import pallas as pl
from jax.experimental.pallas import tpu as pltpu
from jax.experimental.pallas import tpu_sc as plsc

VOCAB = 100000
DIM = 128
BATCH = 1024
P_LEN = 200
Q_LEN = 50

N_ROWS = BATCH * (P_LEN + Q_LEN)
G = 80
CH = 400
NBUF = 2
BB = 64


@functools.cache
def _make_sc_gather(n_rows):
    info = plsc.get_sparse_core_info()
    nc, ns = info.num_cores, info.num_subcores
    nw = nc * ns
    per_w = n_rows // nw
    n_outer = per_w // CH
    n_inner = CH // G
    mesh = plsc.VectorSubcoreMesh(core_axis_name="c", subcore_axis_name="s")

    @functools.partial(
        pl.kernel,
        mesh=mesh,
        out_type=jax.ShapeDtypeStruct((n_rows, DIM), jnp.float32),
        scratch_types=[pltpu.VMEM((per_w,), jnp.int32)]
        + [pltpu.VMEM((CH, DIM), jnp.float32)] * NBUF
        + [pltpu.SemaphoreType.DMA],
    )
    def gather_rows(table_hbm, idx_hbm, out_hbm, idx_v, *bufs_sem):
        bufs, sem = bufs_sem[:-1], bufs_sem[-1]
        wid = lax.axis_index("s") * nc + lax.axis_index("c")
        pltpu.sync_copy(idx_hbm.at[pl.ds(wid * per_w, per_w)], idx_v)

        def fire(i):
            return [
                pltpu.async_copy(
                    table_hbm.at[idx_v.at[pl.ds(i * CH + j * G, G)]],
                    bufs[i % NBUF].at[pl.ds(j * G, G)],
                    sem)
                for j in range(n_inner)
            ]

        in_flight = fire(0)
        for i in range(n_outer):
            for c in in_flight:
                c.wait()
            if i + 1 < n_outer:
                in_flight = fire(i + 1)
            pltpu.sync_copy(bufs[i % NBUF],
                            out_hbm.at[pl.ds(wid * per_w + i * CH, CH)])

    return gather_rows


_HIGH = jax.lax.Precision.HIGHEST


def _tc_body(p_ref, q_ref, ep_ref, eq_ref, ws_ref, we_ref, bs_ref, be_ref,
             sl_ref, el_ref, lss_ref, lse_ref):
    f32 = jnp.float32
    pid = p_ref[...]
    qid = q_ref[...]
    ep2 = ep_ref[...]
    eq2 = eq_ref[...]
    ws = ws_ref[...]
    we = we_ref[...]

    qmask = (qid != 0).astype(f32)
    ones_q = jnp.ones((1, Q_LEN), f32)
    qlen = lax.dot_general(qmask, ones_q, (((1,), (1,)), ((), ())),
                           precision=_HIGH)
    qsum = jnp.sum(eq2.reshape(BB, Q_LEN, DIM), axis=1)
    qbar = qsum / qlen

    ws1, ws2, ws3 = ws[:, :DIM], ws[:, DIM:2 * DIM], ws[:, 2 * DIM:]
    we1, we2, we3 = we[:, :DIM], we[:, DIM:2 * DIM], we[:, 2 * DIM:]
    us = ws1 + ws3 * qbar
    ue = we1 + we3 * qbar
    uu = jnp.concatenate([us[:, None, :], ue[:, None, :]], axis=1)
    cs = jnp.sum(qbar * ws2, axis=1, keepdims=True) + bs_ref[0, 0]
    ce = jnp.sum(qbar * we2, axis=1, keepdims=True) + be_ref[0, 0]
    m = lax.dot_general(uu, ep2.reshape(BB, P_LEN, DIM),
                        (((2,), (2,)), ((0,), (0,))),
                        preferred_element_type=f32)
    raw_s = m[:, 0, :] + cs
    raw_e = m[:, 1, :] + ce

    pm = pid != 0
    ones_p = jnp.ones((1, P_LEN), f32)
    log_eps = jnp.log(ws[0:1, 0:1] * 0.0 + 1e-45)

    def masked(raw):
        lg = jnp.where(pm, raw, -10000000.0)
        t = jnp.where(pm, lg, lg + log_eps)
        tmax = jnp.max(t, axis=1, keepdims=True)
        sh = t - tmax
        se = lax.dot_general(jnp.exp(sh), ones_p, (((1,), (1,)), ((), ())),
                             precision=_HIGH)
        lsm = sh - jnp.log(se)
        return lg, lsm

    sl_ref[...], lss_ref[...] = masked(raw_s)
    el_ref[...], lse_ref[...] = masked(raw_e)


def _tc_call(passage, question, gathered, W_start, b_start, W_end, b_end):
    nb = passage.shape[0]
    grid = (nb // BB,)
    fo = jax.ShapeDtypeStruct((nb, P_LEN), jnp.float32)
    bmap = lambda i: (i, 0)
    zmap = lambda i: (0, 0)
    q_row0 = nb * P_LEN // (BB * Q_LEN)
    return pl.pallas_call(
        _tc_body,
        grid=grid,
        in_specs=[
            pl.BlockSpec((BB, P_LEN), bmap),
            pl.BlockSpec((BB, Q_LEN), bmap),
            pl.BlockSpec((BB * P_LEN, DIM), lambda i: (i, 0)),
            pl.BlockSpec((BB * Q_LEN, DIM), lambda i: (q_row0 + i, 0)),
            pl.BlockSpec((1, 3 * DIM), zmap),
            pl.BlockSpec((1, 3 * DIM), zmap),
            pl.BlockSpec((1, 1), zmap),
            pl.BlockSpec((1, 1), zmap),
        ],
        out_specs=[pl.BlockSpec((BB, P_LEN), bmap)] * 4,
        out_shape=[fo, fo, fo, fo],
    )(passage, question, gathered, gathered, W_start, W_end,
      b_start.reshape(1, 1), b_end.reshape(1, 1))


NCHUNK = 2


def kernel(passage, question, table, W_start, b_start, W_end, b_end):
    cb = BATCH // NCHUNK
    gather = _make_sc_gather(cb * (P_LEN + Q_LEN))
    chunks, gs = [], []
    for k in range(NCHUNK):
        psl = lax.slice_in_dim(passage, k * cb, (k + 1) * cb)
        qsl = lax.slice_in_dim(question, k * cb, (k + 1) * cb)
        idx = jnp.concatenate([psl.reshape(-1), qsl.reshape(-1)])
        chunks.append((psl, qsl))
        gs.append(gather(table, idx))
    parts = [_tc_call(psl, qsl, g, W_start, b_start, W_end, b_end)
             for (psl, qsl), g in zip(chunks, gs)]
    return tuple(jnp.concatenate(p, axis=0) for p in zip(*parts))


# speedup vs baseline: 1.0162x; 1.0162x over previous
import functools

import jax
import jax.numpy as jnp
from jax import lax
from jax.experimental import pallas as pl
from jax.experimental.pallas import tpu as pltpu
from jax.experimental.pallas import tpu_sc as plsc

VOCAB = 100000
DIM = 128
BATCH = 1024
P_LEN = 200
Q_LEN = 50

N_ROWS = BATCH * (P_LEN + Q_LEN)
G = 80
CH = 400
NBUF = 2
BB = 128


@functools.cache
def _make_sc_gather(n_rows):
    info = plsc.get_sparse_core_info()
    nc, ns = info.num_cores, info.num_subcores
    nw = nc * ns
    per_w = n_rows // nw
    n_outer = per_w // CH
    n_inner = CH // G
    mesh = plsc.VectorSubcoreMesh(core_axis_name="c", subcore_axis_name="s")

    @functools.partial(
        pl.kernel,
        mesh=mesh,
        out_type=jax.ShapeDtypeStruct((n_rows, DIM), jnp.float32),
        scratch_types=[pltpu.VMEM((per_w,), jnp.int32)]
        + [pltpu.VMEM((CH, DIM), jnp.float32)] * NBUF
        + [pltpu.SemaphoreType.DMA],
    )
    def gather_rows(table_hbm, idx_hbm, out_hbm, idx_v, *bufs_sem):
        bufs, sem = bufs_sem[:-1], bufs_sem[-1]
        wid = lax.axis_index("s") * nc + lax.axis_index("c")
        pltpu.sync_copy(idx_hbm.at[pl.ds(wid * per_w, per_w)], idx_v)

        def fire(i):
            return [
                pltpu.async_copy(
                    table_hbm.at[idx_v.at[pl.ds(i * CH + j * G, G)]],
                    bufs[i % NBUF].at[pl.ds(j * G, G)],
                    sem)
                for j in range(n_inner)
            ]

        in_flight = fire(0)
        for i in range(n_outer):
            for c in in_flight:
                c.wait()
            if i + 1 < n_outer:
                in_flight = fire(i + 1)
            pltpu.sync_copy(bufs[i % NBUF],
                            out_hbm.at[pl.ds(wid * per_w + i * CH, CH)])

    return gather_rows


_HIGH = jax.lax.Precision.HIGHEST


def _tc_body(p_ref, q_ref, ep_ref, eq_ref, ws_ref, we_ref, bs_ref, be_ref,
             sl_ref, el_ref, lss_ref, lse_ref):
    f32 = jnp.float32
    pid = p_ref[...]
    qid = q_ref[...]
    ep2 = ep_ref[...]
    eq2 = eq_ref[...]
    ws = ws_ref[...]
    we = we_ref[...]

    qmask = (qid != 0).astype(f32)
    ones_q = jnp.ones((1, Q_LEN), f32)
    qlen = lax.dot_general(qmask, ones_q, (((1,), (1,)), ((), ())),
                           precision=_HIGH)
    qsum = jnp.sum(eq2.reshape(BB, Q_LEN, DIM), axis=1)
    qbar = qsum / qlen

    ws1, ws2, ws3 = ws[:, :DIM], ws[:, DIM:2 * DIM], ws[:, 2 * DIM:]
    we1, we2, we3 = we[:, :DIM], we[:, DIM:2 * DIM], we[:, 2 * DIM:]
    us = ws1 + ws3 * qbar
    ue = we1 + we3 * qbar
    uu = jnp.concatenate([us[:, None, :], ue[:, None, :]], axis=1)
    cs = jnp.sum(qbar * ws2, axis=1, keepdims=True) + bs_ref[0, 0]
    ce = jnp.sum(qbar * we2, axis=1, keepdims=True) + be_ref[0, 0]
    m = lax.dot_general(uu, ep2.reshape(BB, P_LEN, DIM),
                        (((2,), (2,)), ((0,), (0,))),
                        preferred_element_type=f32)
    raw_s = m[:, 0, :] + cs
    raw_e = m[:, 1, :] + ce

    pm = pid != 0
    ones_p = jnp.ones((1, P_LEN), f32)
    log_eps = jnp.log(ws[0:1, 0:1] * 0.0 + 1e-45)

    def masked(raw):
        lg = jnp.where(pm, raw, -10000000.0)
        t = jnp.where(pm, lg, lg + log_eps)
        tmax = jnp.max(t, axis=1, keepdims=True)
        sh = t - tmax
        se = lax.dot_general(jnp.exp(sh), ones_p, (((1,), (1,)), ((), ())),
                             precision=_HIGH)
        lsm = sh - jnp.log(se)
        return lg, lsm

    sl_ref[...], lss_ref[...] = masked(raw_s)
    el_ref[...], lse_ref[...] = masked(raw_e)


def _tc_call(passage, question, gathered, W_start, b_start, W_end, b_end):
    nb = passage.shape[0]
    grid = (nb // BB,)
    fo = jax.ShapeDtypeStruct((nb, P_LEN), jnp.float32)
    bmap = lambda i: (i, 0)
    zmap = lambda i: (0, 0)
    q_row0 = nb * P_LEN // (BB * Q_LEN)
    return pl.pallas_call(
        _tc_body,
        grid=grid,
        in_specs=[
            pl.BlockSpec((BB, P_LEN), bmap),
            pl.BlockSpec((BB, Q_LEN), bmap),
            pl.BlockSpec((BB * P_LEN, DIM), lambda i: (i, 0)),
            pl.BlockSpec((BB * Q_LEN, DIM), lambda i: (q_row0 + i, 0)),
            pl.BlockSpec((1, 3 * DIM), zmap),
            pl.BlockSpec((1, 3 * DIM), zmap),
            pl.BlockSpec((1, 1), zmap),
            pl.BlockSpec((1, 1), zmap),
        ],
        out_specs=[pl.BlockSpec((BB, P_LEN), bmap)] * 4,
        out_shape=[fo, fo, fo, fo],
    )(passage, question, gathered, gathered, W_start, W_end,
      b_start.reshape(1, 1), b_end.reshape(1, 1))


NCHUNK = 2


def kernel(passage, question, table, W_start, b_start, W_end, b_end):
    cb = BATCH // NCHUNK
    gather = _make_sc_gather(cb * (P_LEN + Q_LEN))
    chunks, gs = [], []
    for k in range(NCHUNK):
        psl = lax.slice_in_dim(passage, k * cb, (k + 1) * cb)
        qsl = lax.slice_in_dim(question, k * cb, (k + 1) * cb)
        idx = jnp.concatenate([psl.reshape(-1), qsl.reshape(-1)])
        chunks.append((psl, qsl))
        gs.append(gather(table, idx))
    parts = [_tc_call(psl, qsl, g, W_start, b_start, W_end, b_end)
             for (psl, qsl), g in zip(chunks, gs)]
    return tuple(jnp.concatenate(p, axis=0) for p in zip(*parts))
